# baseline (speedup 1.0000x reference)
import jax
import jax.numpy as jnp
from jax import lax
from jax.experimental import pallas as pl
from jax.experimental.pallas import tpu as pltpu
from jax.experimental.pallas import tpu_sc as plsc

NC = 2
NS = 16
NW = NC * NS
L = 16
B = 16384
BPW = B // NW
NQ = 8
Q = BPW // NQ


def _embed_body(src_hbm, lnk_hbm, nodes_hbm, x_hbm, y_hbm, out_hbm,
                src_i, lnk_i, vals, dx, dy, out_v, sem_s, sem_l, sem_o, *sem_q):
    wid = lax.axis_index("s") * NC + lax.axis_index("c")
    base = wid * BPW
    c_src = pltpu.async_copy(src_hbm.at[pl.ds(base, BPW)], src_i, sem_s)
    c_lnk = pltpu.async_copy(lnk_hbm.at[pl.ds(base, BPW)], lnk_i, sem_l)

    c_src.wait()
    c_lnk.wait()
    g = []
    for q in range(NQ):
        sl = pl.ds(q * Q, Q)
        g.append((pltpu.async_copy(nodes_hbm.at[src_i.at[sl]],
                                   vals.at[sl], sem_q[q]),
                  pltpu.async_copy(x_hbm.at[lnk_i.at[sl]],
                                   dx.at[sl], sem_q[q]),
                  pltpu.async_copy(y_hbm.at[lnk_i.at[sl]],
                                   dy.at[sl], sem_q[q])))

    outs = []
    for q in range(NQ):
        for c in g[q]:
            c.wait()
        for i in range(q * Q // L, (q + 1) * Q // L):
            s = pl.ds(i * L, L)
            out_v[s] = dx[s] + dy[s] * vals[s]
        outs.append(pltpu.async_copy(out_v.at[pl.ds(q * Q, Q)],
                                     out_hbm.at[pl.ds(base + q * Q, Q)], sem_o))
    for o in outs:
        o.wait()


def kernel(src, lnk, nodes, x, y):
    mesh = plsc.VectorSubcoreMesh(
        core_axis_name="c", subcore_axis_name="s",
        num_cores=NC, num_subcores=NS)
    f = pl.kernel(
        _embed_body,
        out_type=jax.ShapeDtypeStruct((B,), jnp.float32),
        mesh=mesh,
        scratch_types=[
            pltpu.VMEM((BPW,), jnp.int32),
            pltpu.VMEM((BPW,), jnp.int32),
            pltpu.VMEM((BPW,), jnp.float32),
            pltpu.VMEM((BPW,), jnp.float32),
            pltpu.VMEM((BPW,), jnp.float32),
            pltpu.VMEM((BPW,), jnp.float32),
            pltpu.SemaphoreType.DMA,
            pltpu.SemaphoreType.DMA,
            pltpu.SemaphoreType.DMA,
        ] + [pltpu.SemaphoreType.DMA] * NQ,
    )
    return f(src.astype(jnp.int32), lnk.astype(jnp.int32), nodes, x, y)


# speedup vs baseline: 1.0251x; 1.0251x over previous
import jax
import jax.numpy as jnp
from jax import lax
from jax.experimental import pallas as pl
from jax.experimental.pallas import tpu as pltpu
from jax.experimental.pallas import tpu_sc as plsc

NC = 2
NS = 16
NW = NC * NS
L = 16
B = 16384
BPW = B // NW
NQ = 4
Q = BPW // NQ


def _embed_body(src_hbm, lnk_hbm, nodes_hbm, x_hbm, y_hbm, out_hbm,
                src_i, lnk_i, vals, dx, dy, out_v, sem_s, sem_l, sem_o, *sem_q):
    wid = lax.axis_index("s") * NC + lax.axis_index("c")
    base = wid * BPW
    c_src = pltpu.async_copy(src_hbm.at[pl.ds(base, BPW)], src_i, sem_s)
    c_lnk = pltpu.async_copy(lnk_hbm.at[pl.ds(base, BPW)], lnk_i, sem_l)

    c_src.wait()
    c_lnk.wait()
    g = []
    for q in range(NQ):
        sl = pl.ds(q * Q, Q)
        g.append((pltpu.async_copy(nodes_hbm.at[src_i.at[sl]],
                                   vals.at[sl], sem_q[q]),
                  pltpu.async_copy(x_hbm.at[lnk_i.at[sl]],
                                   dx.at[sl], sem_q[q]),
                  pltpu.async_copy(y_hbm.at[lnk_i.at[sl]],
                                   dy.at[sl], sem_q[q])))

    outs = []
    for q in range(NQ):
        for c in g[q]:
            c.wait()
        for i in range(q * Q // L, (q + 1) * Q // L):
            s = pl.ds(i * L, L)
            out_v[s] = dx[s] + dy[s] * vals[s]
        outs.append(pltpu.async_copy(out_v.at[pl.ds(q * Q, Q)],
                                     out_hbm.at[pl.ds(base + q * Q, Q)], sem_o))
    for o in outs:
        o.wait()


def kernel(src, lnk, nodes, x, y):
    mesh = plsc.VectorSubcoreMesh(
        core_axis_name="c", subcore_axis_name="s",
        num_cores=NC, num_subcores=NS)
    f = pl.kernel(
        _embed_body,
        out_type=jax.ShapeDtypeStruct((B,), jnp.float32),
        mesh=mesh,
        scratch_types=[
            pltpu.VMEM((BPW,), jnp.int32),
            pltpu.VMEM((BPW,), jnp.int32),
            pltpu.VMEM((BPW,), jnp.float32),
            pltpu.VMEM((BPW,), jnp.float32),
            pltpu.VMEM((BPW,), jnp.float32),
            pltpu.VMEM((BPW,), jnp.float32),
            pltpu.SemaphoreType.DMA,
            pltpu.SemaphoreType.DMA,
            pltpu.SemaphoreType.DMA,
        ] + [pltpu.SemaphoreType.DMA] * NQ,
    )
    return f(src.astype(jnp.int32), lnk.astype(jnp.int32), nodes, x, y)
